# baseline (speedup 1.0000x reference)
import jax
import jax.numpy as jnp
from jax import lax
from jax.experimental import pallas as pl
from jax.experimental.pallas import tpu as pltpu
from jax.experimental.pallas import tpu_sc as plsc

_NUM_CORES = 2
_NUM_SUBCORES = 16
_NUM_WORKERS = _NUM_CORES * _NUM_SUBCORES
_CHUNK_ROWS = 16
_NBUF = 3
_SPLIT = 3072
_BLOCK_ROWS = 512


def _sc_body(emb_hbm, out_hbm, *scratch):
    bufs = list(scratch[:_NBUF])
    isems = list(scratch[_NBUF : 2 * _NBUF])
    osems = list(scratch[2 * _NBUF : 3 * _NBUF])
    wid = lax.axis_index("s") * _NUM_CORES + lax.axis_index("c")
    rows = out_hbm.shape[0] // _NUM_WORKERS
    base = wid * rows
    nchunks = rows // _CHUNK_ROWS
    in_c = [None] * _NBUF
    out_c = [None] * _NBUF
    for i in range(nchunks):
        b = i % _NBUF
        if out_c[b] is not None:
            out_c[b].wait()
        lo = base + i * _CHUNK_ROWS
        in_c[b] = pltpu.async_copy(emb_hbm.at[pl.ds(lo, _CHUNK_ROWS)], bufs[b], isems[b])
        if i > 0:
            pb = (i - 1) % _NBUF
            in_c[pb].wait()
            plo = base + (i - 1) * _CHUNK_ROWS
            out_c[pb] = pltpu.async_copy(bufs[pb], out_hbm.at[pl.ds(plo, _CHUNK_ROWS)], osems[pb])
    lb = (nchunks - 1) % _NBUF
    in_c[lb].wait()
    llo = base + (nchunks - 1) * _CHUNK_ROWS
    out_c[lb] = pltpu.async_copy(bufs[lb], out_hbm.at[pl.ds(llo, _CHUNK_ROWS)], osems[lb])
    for b in range(_NBUF):
        if out_c[b] is not None:
            out_c[b].wait()


def _tc_body(emb_ref, out_ref):
    out_ref[...] = emb_ref[...]


def kernel(x, emb):
    seq_len = x.shape[1]
    d = emb.shape[1]
    mesh = plsc.VectorSubcoreMesh(core_axis_name="c", subcore_axis_name="s")
    out_sc = pl.kernel(
        _sc_body,
        out_type=jax.ShapeDtypeStruct((_SPLIT, d), emb.dtype),
        mesh=mesh,
        scratch_types=(
            [pltpu.VMEM((_CHUNK_ROWS, d), jnp.float32)] * _NBUF
            + [pltpu.SemaphoreType.DMA] * (2 * _NBUF)
        ),
    )(emb[:_SPLIT])
    rest = seq_len - _SPLIT
    out_tc = pl.pallas_call(
        _tc_body,
        grid=(rest // _BLOCK_ROWS,),
        in_specs=[pl.BlockSpec((_BLOCK_ROWS, d), lambda i: (i, 0))],
        out_specs=pl.BlockSpec((_BLOCK_ROWS, d), lambda i: (i, 0)),
        out_shape=jax.ShapeDtypeStruct((rest, d), emb.dtype),
    )(lax.slice(emb, (_SPLIT, 0), (seq_len, d)))
    return jnp.concatenate([out_sc, out_tc], axis=0)[None]


# speedup vs baseline: 1.4511x; 1.4511x over previous
import jax
import jax.numpy as jnp
from jax import lax
from jax.experimental import pallas as pl
from jax.experimental.pallas import tpu as pltpu
from jax.experimental.pallas import tpu_sc as plsc

_NUM_CORES = 2
_NUM_SUBCORES = 16
_NUM_WORKERS = _NUM_CORES * _NUM_SUBCORES
_CHUNK_ROWS = 16
_NBUF = 3
_SPLIT = 3072
_BLOCK_ROWS = 512


def _sc_body(emb_hbm, out_hbm, *scratch):
    bufs = list(scratch[:_NBUF])
    isems = list(scratch[_NBUF : 2 * _NBUF])
    osems = list(scratch[2 * _NBUF : 3 * _NBUF])
    wid = lax.axis_index("s") * _NUM_CORES + lax.axis_index("c")
    rows = out_hbm.shape[0] // _NUM_WORKERS
    base = wid * rows
    nchunks = rows // _CHUNK_ROWS
    in_c = [None] * _NBUF
    out_c = [None] * _NBUF
    for i in range(nchunks):
        b = i % _NBUF
        if out_c[b] is not None:
            out_c[b].wait()
        lo = base + i * _CHUNK_ROWS
        in_c[b] = pltpu.async_copy(emb_hbm.at[pl.ds(lo, _CHUNK_ROWS)], bufs[b], isems[b])
        if i > 0:
            pb = (i - 1) % _NBUF
            in_c[pb].wait()
            plo = base + (i - 1) * _CHUNK_ROWS
            out_c[pb] = pltpu.async_copy(bufs[pb], out_hbm.at[pl.ds(plo, _CHUNK_ROWS)], osems[pb])
    lb = (nchunks - 1) % _NBUF
    in_c[lb].wait()
    llo = base + (nchunks - 1) * _CHUNK_ROWS
    out_c[lb] = pltpu.async_copy(bufs[lb], out_hbm.at[pl.ds(llo, _CHUNK_ROWS)], osems[lb])
    for b in range(_NBUF):
        if out_c[b] is not None:
            out_c[b].wait()


def _tc_body(emb_ref, out_ref):
    out_ref[...] = emb_ref[...]


def kernel(x, emb):
    seq_len = x.shape[1]
    d = emb.shape[1]
    mesh = plsc.VectorSubcoreMesh(core_axis_name="c", subcore_axis_name="s")
    out_sc = pl.kernel(
        _sc_body,
        out_type=jax.ShapeDtypeStruct((_SPLIT, d), emb.dtype),
        mesh=mesh,
        scratch_types=(
            [pltpu.VMEM((_CHUNK_ROWS, d), jnp.float32)] * _NBUF
            + [pltpu.SemaphoreType.DMA] * (2 * _NBUF)
        ),
    )(emb)
    rest = seq_len - _SPLIT
    out_tc = pl.pallas_call(
        _tc_body,
        grid=(rest // _BLOCK_ROWS,),
        in_specs=[pl.BlockSpec((_BLOCK_ROWS, d), lambda i: (i + _SPLIT // _BLOCK_ROWS, 0))],
        out_specs=pl.BlockSpec((_BLOCK_ROWS, d), lambda i: (i, 0)),
        out_shape=jax.ShapeDtypeStruct((rest, d), emb.dtype),
    )(emb)
    return jnp.concatenate([out_sc, out_tc], axis=0)[None]


# speedup vs baseline: 2.2009x; 1.5167x over previous
import jax
import jax.numpy as jnp
from jax import lax
from jax.experimental import pallas as pl
from jax.experimental.pallas import tpu as pltpu
from jax.experimental.pallas import tpu_sc as plsc

_NUM_CORES = 2
_NUM_SUBCORES = 16
_NUM_WORKERS = _NUM_CORES * _NUM_SUBCORES
_CHUNK_ROWS = 16
_NBUF = 3
_SC_ROWS = 4096
_BLOCK_ROWS = 512


def _sc_body(emb_hbm, out_hbm, *scratch):
    bufs = list(scratch[:_NBUF])
    isems = list(scratch[_NBUF : 2 * _NBUF])
    osems = list(scratch[2 * _NBUF : 3 * _NBUF])
    wid = lax.axis_index("s") * _NUM_CORES + lax.axis_index("c")
    rows = _SC_ROWS // _NUM_WORKERS
    base = wid * rows
    nchunks = rows // _CHUNK_ROWS
    in_c = [None] * _NBUF
    out_c = [None] * _NBUF
    for i in range(nchunks):
        b = i % _NBUF
        if out_c[b] is not None:
            out_c[b].wait()
        lo = base + i * _CHUNK_ROWS
        in_c[b] = pltpu.async_copy(
            emb_hbm.at[pl.ds(lo, _CHUNK_ROWS)], bufs[b], isems[b]
        )
        if i > 0:
            pb = (i - 1) % _NBUF
            in_c[pb].wait()
            plo = base + (i - 1) * _CHUNK_ROWS
            out_c[pb] = pltpu.async_copy(
                bufs[pb], out_hbm.at[pl.ds(plo, _CHUNK_ROWS)], osems[pb]
            )
    lb = (nchunks - 1) % _NBUF
    in_c[lb].wait()
    llo = base + (nchunks - 1) * _CHUNK_ROWS
    out_c[lb] = pltpu.async_copy(
        bufs[lb], out_hbm.at[pl.ds(llo, _CHUNK_ROWS)], osems[lb]
    )
    for b in range(_NBUF):
        if out_c[b] is not None:
            out_c[b].wait()


def _tc_body(carry_ref, emb_ref, out_ref):
    del carry_ref
    out_ref[...] = emb_ref[...]


def kernel(x, emb):
    seq_len = x.shape[1]
    d = emb.shape[1]
    mesh = plsc.VectorSubcoreMesh(core_axis_name="c", subcore_axis_name="s")
    out_sc = pl.kernel(
        _sc_body,
        out_type=jax.ShapeDtypeStruct((seq_len, d), emb.dtype),
        mesh=mesh,
        scratch_types=(
            [pltpu.VMEM((_CHUNK_ROWS, d), jnp.float32)] * _NBUF
            + [pltpu.SemaphoreType.DMA] * (2 * _NBUF)
        ),
    )(emb)
    base_blocks = _SC_ROWS // _BLOCK_ROWS
    out = pl.pallas_call(
        _tc_body,
        grid=((seq_len - _SC_ROWS) // _BLOCK_ROWS,),
        in_specs=[
            pl.BlockSpec(memory_space=pl.ANY),
            pl.BlockSpec((_BLOCK_ROWS, d), lambda i: (i + base_blocks, 0)),
        ],
        out_specs=pl.BlockSpec((_BLOCK_ROWS, d), lambda i: (i + base_blocks, 0)),
        out_shape=jax.ShapeDtypeStruct((seq_len, d), emb.dtype),
        input_output_aliases={0: 0},
    )(out_sc, emb)
    return out[None]
